# Initial kernel scaffold; baseline (speedup 1.0000x reference)
import jax
import jax.numpy as jnp
from jax.experimental import pallas as pl


def kernel(x, edge_index, W1, W2, W3):
    raise NotImplementedError("write your pallas kernel here")


# speedup vs baseline: 7.8759x; 7.8759x over previous
import functools

import jax
import jax.numpy as jnp
from jax import lax
from jax.experimental import pallas as pl
from jax.experimental.pallas import tpu as pltpu
from jax.experimental.pallas import tpu_sc as plsc

N = 10000
D = 128
NPAD = 10240
E = 320000
NC, NS = 2, 16
NW = NC * NS
EW = 10240
EPAD = NW * EW
C = 128
NCHUNK = EW // C
RPT = NPAD // NS
DN = 16
PAD_ROW = NPAD - 1
PAD_COL = NPAD - 2

_mesh = plsc.VectorSubcoreMesh(
    core_axis_name="c", subcore_axis_name="s", num_cores=NC, num_subcores=NS)


def _make_prop(width):
  @functools.partial(
      pl.kernel,
      out_type=jax.ShapeDtypeStruct((NC, NPAD, width), jnp.float32),
      mesh=_mesh,
      compiler_params=pltpu.CompilerParams(use_tc_tiling_on_sc=(width == D)),
      scratch_types=[
          pltpu.VMEM((C,), jnp.int32),
          pltpu.VMEM((C,), jnp.int32),
          pltpu.VMEM((C, width), jnp.float32),
          pltpu.VMEM_SHARED((NPAD, width), jnp.float32),
          pltpu.SemaphoreType.DMA,
      ],
  )
  def prop(row_hbm, col_hbm, z_hbm, out_hbm, row_v, col_v, gath_v, acc, sem):
    cid = lax.axis_index("c")
    sid = lax.axis_index("s")
    wid = sid * NC + cid

    def zrow(i, carry):
      for j in range(width // 16):
        gath_v[i, pl.ds(j * 16, 16)] = jnp.zeros((16,), jnp.float32)
      return carry
    lax.fori_loop(0, C, zrow, 0)

    def zacc(k, carry):
      pltpu.sync_copy(gath_v, acc.at[pl.ds(sid * RPT + k * C, C)])
      return carry
    lax.fori_loop(0, RPT // C, zacc, 0)
    plsc.subcore_barrier()

    base = wid * EW
    def chunk(g, carry):
      off = base + g * C
      pltpu.sync_copy(row_hbm.at[pl.ds(off, C)], row_v)
      pltpu.sync_copy(col_hbm.at[pl.ds(off, C)], col_v)
      pltpu.async_copy(z_hbm.at[row_v], gath_v, sem).wait()
      pltpu.sync_copy(gath_v, acc.at[col_v], add=True)
      return carry
    lax.fori_loop(0, NCHUNK, chunk, 0)
    plsc.subcore_barrier()

    def wout(k, carry):
      r0 = sid * RPT + k * C
      pltpu.sync_copy(acc.at[pl.ds(r0, C)], out_hbm.at[cid, pl.ds(r0, C)])
      return carry
    lax.fori_loop(0, RPT // C, wout, 0)

  return prop


_prop_wide = _make_prop(D)
_prop_narrow = _make_prop(DN)


@functools.partial(
    pl.kernel,
    out_type=jax.ShapeDtypeStruct((NC, NPAD, DN), jnp.float32),
    mesh=_mesh,
    compiler_params=pltpu.CompilerParams(use_tc_tiling_on_sc=False),
    scratch_types=[
        pltpu.VMEM((C,), jnp.int32),
        pltpu.VMEM((C, DN), jnp.float32),
        pltpu.VMEM((C, DN), jnp.float32),
        pltpu.VMEM_SHARED((NPAD, DN), jnp.float32),
    ],
)
def _degree(col_hbm, out_hbm, col_v, ones_v, zero_v, acc):
  cid = lax.axis_index("c")
  sid = lax.axis_index("s")
  wid = sid * NC + cid

  def fill(i, carry):
    ones_v[i, pl.ds(0, DN)] = jnp.ones((DN,), jnp.float32)
    zero_v[i, pl.ds(0, DN)] = jnp.zeros((DN,), jnp.float32)
    return carry
  lax.fori_loop(0, C, fill, 0)

  def zacc(k, carry):
    pltpu.sync_copy(zero_v, acc.at[pl.ds(sid * RPT + k * C, C)])
    return carry
  lax.fori_loop(0, RPT // C, zacc, 0)
  plsc.subcore_barrier()

  base = wid * EW
  def chunk(g, carry):
    pltpu.sync_copy(col_hbm.at[pl.ds(base + g * C, C)], col_v)
    pltpu.sync_copy(ones_v, acc.at[col_v], add=True)
    return carry
  lax.fori_loop(0, NCHUNK, chunk, 0)
  plsc.subcore_barrier()

  def wout(k, carry):
    r0 = sid * RPT + k * C
    pltpu.sync_copy(acc.at[pl.ds(r0, C)], out_hbm.at[cid, pl.ds(r0, C)])
    return carry
  lax.fori_loop(0, RPT // C, wout, 0)


_BR = 1024


def _tc_call(body, n_out, out_widths, in_specs):
  grid = NPAD // _BR
  return pl.pallas_call(
      body,
      grid=(grid,),
      in_specs=in_specs,
      out_specs=[pl.BlockSpec((_BR, w), lambda i: (i, 0)) for w in out_widths],
      out_shape=[jax.ShapeDtypeStruct((NPAD, w), jnp.float32)
                 for w in out_widths],
  )


def _rowspec(w):
  return pl.BlockSpec((_BR, w), lambda i: (i, 0))


def _fullspec(a, b):
  return pl.BlockSpec((a, b), lambda i: (0, 0))


def _tc1_body(x_ref, d0_ref, d1_ref, w_ref, z_ref, dinv_ref):
  dinv = lax.rsqrt(d0_ref[...] + d1_ref[...] + 1.0)
  dinv_ref[...] = dinv
  z_ref[...] = dinv * jnp.dot(x_ref[...], w_ref[...],
                              preferred_element_type=jnp.float32)


def _tc2_body(p0_ref, p1_ref, z_ref, dinv_ref, w_ref, out_ref):
  dinv = dinv_ref[...]
  s = jnp.maximum(dinv * (p0_ref[...] + p1_ref[...] + z_ref[...]), 0.0)
  out_ref[...] = dinv * jnp.dot(s, w_ref[...],
                                preferred_element_type=jnp.float32)


def _tc3_body(p0_ref, p1_ref, z_ref, dinv_ref, w_ref, out_ref):
  dinv = dinv_ref[...]
  s = jnp.maximum(dinv * (p0_ref[...] + p1_ref[...] + z_ref[...]), 0.0)
  z3 = dinv * jnp.dot(s, w_ref[...], preferred_element_type=jnp.float32)
  lane = lax.broadcasted_iota(jnp.int32, (1, DN), 1)
  out_ref[...] = z3 * (lane == 0).astype(jnp.float32)


def _tc4_body(t0_ref, t1_ref, z3_ref, dinv_ref, out_ref):
  out_ref[...] = dinv_ref[...] * (
      t0_ref[...][:, :1] + t1_ref[...][:, :1] + z3_ref[...][:, :1])


def kernel(x, edge_index, W1, W2, W3):
  row = edge_index[0].astype(jnp.int32)
  col = edge_index[1].astype(jnp.int32)
  npad_e = EPAD - E
  row = jnp.concatenate([row, jnp.full((npad_e,), PAD_ROW, jnp.int32)])
  col = jnp.concatenate([col, jnp.full((npad_e,), PAD_COL, jnp.int32)])
  xp = jnp.zeros((NPAD, D), jnp.float32).at[:N].set(x)

  degp = _degree(col)
  d0 = degp[0, :, 0].reshape(NPAD, 1)
  d1 = degp[1, :, 0].reshape(NPAD, 1)

  z1, dinv = _tc_call(
      _tc1_body, 2, (D, 1),
      [_rowspec(D), _rowspec(1), _rowspec(1), _fullspec(D, D)],
  )(xp, d0, d1, W1)

  p = _prop_wide(row, col, z1)
  (z2,) = _tc_call(
      _tc2_body, 1, (D,),
      [_rowspec(D), _rowspec(D), _rowspec(D), _rowspec(1), _fullspec(D, D)],
  )(p[0], p[1], z1, dinv, W2)

  q = _prop_wide(row, col, z2)
  (z3w,) = _tc_call(
      _tc3_body, 1, (DN,),
      [_rowspec(D), _rowspec(D), _rowspec(D), _rowspec(1), _fullspec(D, 1)],
  )(q[0], q[1], z2, dinv, W3)

  t = _prop_narrow(row, col, z3w)
  (outp,) = _tc_call(
      _tc4_body, 1, (1,),
      [_rowspec(DN), _rowspec(DN), _rowspec(DN), _rowspec(1)],
  )(t[0], t[1], z3w, dinv)

  return outp[:N]


# speedup vs baseline: 9.8618x; 1.2522x over previous
import functools

import jax
import jax.numpy as jnp
from jax import lax
from jax.experimental import pallas as pl
from jax.experimental.pallas import tpu as pltpu
from jax.experimental.pallas import tpu_sc as plsc

N = 10000
D = 128
NPAD = 10240
E = 320000
NC, NS = 2, 16
NW = NC * NS
EW = 10240
EPAD = NW * EW
C = 128
NCHUNK = EW // C
RPT = NPAD // NS
DN = 16
PAD_ROW = NPAD - 1
PAD_COL = NPAD - 2

_mesh = plsc.VectorSubcoreMesh(
    core_axis_name="c", subcore_axis_name="s", num_cores=NC, num_subcores=NS)


def _make_prop(width):
  @functools.partial(
      pl.kernel,
      out_type=jax.ShapeDtypeStruct((NC, NPAD, width), jnp.float32),
      mesh=_mesh,
      compiler_params=pltpu.CompilerParams(use_tc_tiling_on_sc=(width == D)),
      scratch_types=[
          pltpu.VMEM((NCHUNK, C), jnp.int32),
          pltpu.VMEM((2, C), jnp.int32),
          pltpu.VMEM((2, C, width), jnp.float32),
          pltpu.VMEM_SHARED((NPAD, width), jnp.float32),
          pltpu.SemaphoreType.DMA,
          pltpu.SemaphoreType.DMA,
          pltpu.SemaphoreType.DMA,
          pltpu.SemaphoreType.DMA,
          pltpu.SemaphoreType.DMA,
          pltpu.SemaphoreType.DMA,
      ],
  )
  def prop(row_hbm, col_hbm, z_hbm, out_hbm, row2d, colbuf, gbuf, acc,
           gsem0, gsem1, ssem0, ssem1, csem0, csem1):
    cid = lax.axis_index("c")
    sid = lax.axis_index("s")
    wid = sid * NC + cid
    gsems = (gsem0, gsem1)
    ssems = (ssem0, ssem1)
    csems = (csem0, csem1)

    pltpu.sync_copy(row_hbm.at[pl.ds(wid * NCHUNK, NCHUNK)], row2d)

    def zrow(i, carry):
      for j in range(width // 16):
        gbuf[0, i, pl.ds(j * 16, 16)] = jnp.zeros((16,), jnp.float32)
      return carry
    lax.fori_loop(0, C, zrow, 0)

    def zacc(k, carry):
      pltpu.sync_copy(gbuf.at[0], acc.at[pl.ds(sid * RPT + k * C, C)])
      return carry
    lax.fori_loop(0, RPT // C, zacc, 0)
    plsc.subcore_barrier()

    base = wid * NCHUNK
    for b in range(2):
      pltpu.async_copy(col_hbm.at[base + b], colbuf.at[b], csems[b])
      pltpu.async_copy(z_hbm.at[row2d.at[b]], gbuf.at[b], gsems[b])

    def pair(k, carry):
      g0 = k * 2
      for b in range(2):
        g = g0 + b
        pltpu.make_async_copy(z_hbm.at[row2d.at[g]], gbuf.at[b],
                              gsems[b]).wait()
        pltpu.make_async_copy(col_hbm.at[base + g], colbuf.at[b],
                              csems[b]).wait()
        pltpu.async_copy(gbuf.at[b], acc.at[colbuf.at[b]], ssems[b],
                         add=True).wait()
        @pl.when(g + 2 < NCHUNK)
        def _():
          pltpu.async_copy(col_hbm.at[base + g + 2], colbuf.at[b], csems[b])
          pltpu.async_copy(z_hbm.at[row2d.at[g + 2]], gbuf.at[b], gsems[b])
      return carry
    lax.fori_loop(0, NCHUNK // 2, pair, 0)
    plsc.subcore_barrier()

    def wout(k, carry):
      r0 = sid * RPT + k * C
      pltpu.sync_copy(acc.at[pl.ds(r0, C)], out_hbm.at[cid, pl.ds(r0, C)])
      return carry
    lax.fori_loop(0, RPT // C, wout, 0)

  return prop


_prop_wide = _make_prop(D)
_prop_narrow = _make_prop(DN)


@functools.partial(
    pl.kernel,
    out_type=jax.ShapeDtypeStruct((NC, NPAD, DN), jnp.float32),
    mesh=_mesh,
    compiler_params=pltpu.CompilerParams(use_tc_tiling_on_sc=False),
    scratch_types=[
        pltpu.VMEM((NCHUNK, C), jnp.int32),
        pltpu.VMEM((C, DN), jnp.float32),
        pltpu.VMEM((C, DN), jnp.float32),
        pltpu.VMEM_SHARED((NPAD, DN), jnp.float32),
    ],
)
def _degree(col_hbm, out_hbm, col2d, ones_v, zero_v, acc):
  cid = lax.axis_index("c")
  sid = lax.axis_index("s")
  wid = sid * NC + cid

  pltpu.sync_copy(col_hbm.at[pl.ds(wid * NCHUNK, NCHUNK)], col2d)

  def fill(i, carry):
    ones_v[i, pl.ds(0, DN)] = jnp.ones((DN,), jnp.float32)
    zero_v[i, pl.ds(0, DN)] = jnp.zeros((DN,), jnp.float32)
    return carry
  lax.fori_loop(0, C, fill, 0)

  def zacc(k, carry):
    pltpu.sync_copy(zero_v, acc.at[pl.ds(sid * RPT + k * C, C)])
    return carry
  lax.fori_loop(0, RPT // C, zacc, 0)
  plsc.subcore_barrier()

  def chunk(g, carry):
    pltpu.sync_copy(ones_v, acc.at[col2d.at[g]], add=True)
    return carry
  lax.fori_loop(0, NCHUNK, chunk, 0)
  plsc.subcore_barrier()

  def wout(k, carry):
    r0 = sid * RPT + k * C
    pltpu.sync_copy(acc.at[pl.ds(r0, C)], out_hbm.at[cid, pl.ds(r0, C)])
    return carry
  lax.fori_loop(0, RPT // C, wout, 0)


_BR = 1024


def _tc_call(body, n_out, out_widths, in_specs):
  grid = NPAD // _BR
  return pl.pallas_call(
      body,
      grid=(grid,),
      in_specs=in_specs,
      out_specs=[pl.BlockSpec((_BR, w), lambda i: (i, 0)) for w in out_widths],
      out_shape=[jax.ShapeDtypeStruct((NPAD, w), jnp.float32)
                 for w in out_widths],
  )


def _rowspec(w):
  return pl.BlockSpec((_BR, w), lambda i: (i, 0))


def _fullspec(a, b):
  return pl.BlockSpec((a, b), lambda i: (0, 0))


def _tc1_body(x_ref, d0_ref, d1_ref, w_ref, z_ref, dinv_ref):
  dinv = lax.rsqrt(d0_ref[...] + d1_ref[...] + 1.0)
  dinv_ref[...] = dinv
  z_ref[...] = dinv * jnp.dot(x_ref[...], w_ref[...],
                              preferred_element_type=jnp.float32)


def _tc2_body(p0_ref, p1_ref, z_ref, dinv_ref, w_ref, out_ref):
  dinv = dinv_ref[...]
  s = jnp.maximum(dinv * (p0_ref[...] + p1_ref[...] + z_ref[...]), 0.0)
  out_ref[...] = dinv * jnp.dot(s, w_ref[...],
                                preferred_element_type=jnp.float32)


def _tc3_body(p0_ref, p1_ref, z_ref, dinv_ref, w_ref, out_ref):
  dinv = dinv_ref[...]
  s = jnp.maximum(dinv * (p0_ref[...] + p1_ref[...] + z_ref[...]), 0.0)
  z3 = dinv * jnp.dot(s, w_ref[...], preferred_element_type=jnp.float32)
  lane = lax.broadcasted_iota(jnp.int32, (1, DN), 1)
  out_ref[...] = z3 * (lane == 0).astype(jnp.float32)


def _tc4_body(t0_ref, t1_ref, z3_ref, dinv_ref, out_ref):
  out_ref[...] = dinv_ref[...] * (
      t0_ref[...][:, :1] + t1_ref[...][:, :1] + z3_ref[...][:, :1])


def kernel(x, edge_index, W1, W2, W3):
  row = edge_index[0].astype(jnp.int32)
  col = edge_index[1].astype(jnp.int32)
  npad_e = EPAD - E
  row = jnp.concatenate([row, jnp.full((npad_e,), PAD_ROW, jnp.int32)])
  col = jnp.concatenate([col, jnp.full((npad_e,), PAD_COL, jnp.int32)])
  row = row.reshape(NW * NCHUNK, C)
  col = col.reshape(NW * NCHUNK, C)
  xp = jnp.zeros((NPAD, D), jnp.float32).at[:N].set(x)

  degp = _degree(col)
  d0 = degp[0, :, 0].reshape(NPAD, 1)
  d1 = degp[1, :, 0].reshape(NPAD, 1)

  z1, dinv = _tc_call(
      _tc1_body, 2, (D, 1),
      [_rowspec(D), _rowspec(1), _rowspec(1), _fullspec(D, D)],
  )(xp, d0, d1, W1)

  p = _prop_wide(row, col, z1)
  (z2,) = _tc_call(
      _tc2_body, 1, (D,),
      [_rowspec(D), _rowspec(D), _rowspec(D), _rowspec(1), _fullspec(D, D)],
  )(p[0], p[1], z1, dinv, W2)

  q = _prop_wide(row, col, z2)
  (z3w,) = _tc_call(
      _tc3_body, 1, (DN,),
      [_rowspec(D), _rowspec(D), _rowspec(D), _rowspec(1), _fullspec(D, 1)],
  )(q[0], q[1], z2, dinv, W3)

  t = _prop_narrow(row, col, z3w)
  (outp,) = _tc_call(
      _tc4_body, 1, (1,),
      [_rowspec(DN), _rowspec(DN), _rowspec(DN), _rowspec(1)],
  )(t[0], t[1], z3w, dinv)

  return outp[:N]
